# Initial kernel scaffold; baseline (speedup 1.0000x reference)
import jax
import jax.numpy as jnp
from jax.experimental import pallas as pl


def kernel(input, edge_index, edge_weight, weight):
    raise NotImplementedError("write your pallas kernel here")


# speedup vs baseline: 6.4743x; 6.4743x over previous
import functools

import jax
import jax.numpy as jnp
from jax import lax
from jax.experimental import pallas as pl
from jax.experimental.pallas import tpu as pltpu
from jax.experimental.pallas import tpu_sc as plsc

N = 10000
E = 320000
D = 128

NC = 2
NS = 16
NW = NC * NS
EPW = E // NW
K = 80
NCHUNK = EPW // K
NPAD = 10240
ROWS_PER_TILE = NPAD // NS
ZROWS = 64
SUP = 25
NSUP = NCHUNK // SUP
L = 16


def _matmul_body(x_ref, w_ref, o_ref):
    o_ref[...] = jnp.dot(x_ref[...], w_ref[...],
                         preferred_element_type=jnp.float32)


def _matmul(x, w):
    BM = 1000
    return pl.pallas_call(
        _matmul_body,
        grid=(N // BM,),
        in_specs=[
            pl.BlockSpec((BM, D), lambda i: (i, 0)),
            pl.BlockSpec((D, D), lambda i: (0, 0)),
        ],
        out_specs=pl.BlockSpec((BM, D), lambda i: (i, 0)),
        out_shape=jax.ShapeDtypeStruct((N, D), jnp.float32),
    )(x, w)


def _add_body(p_ref, o_ref):
    o_ref[...] = p_ref[0] + p_ref[1]


def _combine(partial):
    BM = 1000
    return pl.pallas_call(
        _add_body,
        grid=(N // BM,),
        in_specs=[pl.BlockSpec((NC, BM, D), lambda i: (0, i, 0))],
        out_specs=pl.BlockSpec((BM, D), lambda i: (i, 0)),
        out_shape=jax.ShapeDtypeStruct((N, D), jnp.float32),
    )(partial)


def _sc_body(support_hbm, src_hbm, dst_hbm, w_hbm, out_hbm,
             src_v, dst_v, w_v, rows_v, zbuf_v, acc_sh, sem):
    c = lax.axis_index("c")
    s = lax.axis_index("s")
    wid = s * NC + c

    zeros = jnp.zeros((L,), jnp.float32)

    def zero_row(i, carry):
        for j in range(D // L):
            zbuf_v[i, pl.ds(L * j, L)] = zeros
        return carry

    lax.fori_loop(0, ZROWS, zero_row, 0)
    for r in range(ROWS_PER_TILE // ZROWS):
        pltpu.sync_copy(zbuf_v,
                        acc_sh.at[pl.ds(s * ROWS_PER_TILE + r * ZROWS, ZROWS)])
    plsc.subcore_barrier()

    def sup_body(sb, scarry):
        pltpu.sync_copy(src_hbm.at[wid, sb], src_v)
        pltpu.sync_copy(dst_hbm.at[wid, sb], dst_v)
        pltpu.sync_copy(w_hbm.at[wid, sb], w_v)

        def chunk_body(ci, carry):
            pltpu.async_copy(support_hbm.at[src_v.at[ci]], rows_v, sem).wait()

            for g in range(K // L):
                wvec = w_v[ci, pl.ds(L * g, L)]
                for l in range(L):
                    e = L * g + l
                    wsplat = wvec.at[jnp.full((L,), l, jnp.int32)].get(
                        mode="promise_in_bounds")
                    for j in range(D // L):
                        sl = pl.ds(L * j, L)
                        rows_v[e, sl] = rows_v[e, sl] * wsplat

            pltpu.sync_copy(rows_v, acc_sh.at[dst_v.at[ci]], add=True)
            return carry

        lax.fori_loop(0, SUP, chunk_body, 0)
        return scarry

    lax.fori_loop(0, NSUP, sup_body, 0)
    plsc.subcore_barrier()

    pltpu.sync_copy(acc_sh.at[pl.ds(s * ROWS_PER_TILE, ROWS_PER_TILE)],
                    out_hbm.at[c, pl.ds(s * ROWS_PER_TILE, ROWS_PER_TILE)])


def _sc_scatter(support, src, dst, w):
    mesh = plsc.VectorSubcoreMesh(core_axis_name="c", subcore_axis_name="s")
    fn = functools.partial(
        pl.kernel,
        mesh=mesh,
        out_type=jax.ShapeDtypeStruct((NC, NPAD, D), jnp.float32),
        scratch_types=[
            pltpu.VMEM((SUP, K), jnp.int32),
            pltpu.VMEM((SUP, K), jnp.int32),
            pltpu.VMEM((SUP, K), jnp.float32),
            pltpu.VMEM((K, D), jnp.float32),
            pltpu.VMEM((ZROWS, D), jnp.float32),
            pltpu.VMEM_SHARED((NPAD, D), jnp.float32),
            pltpu.SemaphoreType.DMA,
        ],
    )(_sc_body)
    return fn(support, src, dst, w)


def kernel(input, edge_index, edge_weight, weight):
    support = _matmul(input, weight)
    src = edge_index[0].reshape(NW, NSUP, SUP, K)
    dst = edge_index[1].reshape(NW, NSUP, SUP, K)
    w = edge_weight.reshape(NW, NSUP, SUP, K)
    partial = _sc_scatter(support, src, dst, w)
    return _combine(partial)


# speedup vs baseline: 8.6339x; 1.3336x over previous
import functools

import jax
import jax.numpy as jnp
from jax import lax
from jax.experimental import pallas as pl
from jax.experimental.pallas import tpu as pltpu
from jax.experimental.pallas import tpu_sc as plsc

N = 10000
E = 320000
D = 128

NC = 2
NS = 16
NW = NC * NS
EPW = E // NW
K = 80
NCHUNK = EPW // K
NPAD = 10240
ROWS_PER_TILE = NPAD // NS
ZROWS = 64
SUP = 25
NSUP = NCHUNK // SUP
L = 16


def _matmul_body(x_ref, w_ref, o_ref):
    o_ref[...] = jnp.dot(x_ref[...], w_ref[...],
                         preferred_element_type=jnp.float32)


def _matmul(x, w):
    BM = 1000
    return pl.pallas_call(
        _matmul_body,
        grid=(N // BM,),
        in_specs=[
            pl.BlockSpec((BM, D), lambda i: (i, 0)),
            pl.BlockSpec((D, D), lambda i: (0, 0)),
        ],
        out_specs=pl.BlockSpec((BM, D), lambda i: (i, 0)),
        out_shape=jax.ShapeDtypeStruct((N, D), jnp.float32),
    )(x, w)


def _add_body(p_ref, o_ref):
    o_ref[...] = p_ref[0] + p_ref[1]


def _combine(partial):
    BM = 1000
    return pl.pallas_call(
        _add_body,
        grid=(N // BM,),
        in_specs=[pl.BlockSpec((NC, BM, D), lambda i: (0, i, 0))],
        out_specs=pl.BlockSpec((BM, D), lambda i: (i, 0)),
        out_shape=jax.ShapeDtypeStruct((N, D), jnp.float32),
    )(partial)


def _sc_body(support_hbm, src_hbm, dst_hbm, w_hbm, out_hbm,
             src_v, dst_v, w_v, rows0_v, rows1_v, zbuf_v, acc_sh,
             sem0, sem1):
    c = lax.axis_index("c")
    s = lax.axis_index("s")
    wid = s * NC + c

    zeros = jnp.zeros((L,), jnp.float32)

    def zero_row(i, carry):
        for j in range(D // L):
            zbuf_v[i, pl.ds(L * j, L)] = zeros
        return carry

    lax.fori_loop(0, ZROWS, zero_row, 0)
    for r in range(ROWS_PER_TILE // ZROWS):
        pltpu.sync_copy(zbuf_v,
                        acc_sh.at[pl.ds(s * ROWS_PER_TILE + r * ZROWS, ZROWS)])
    plsc.subcore_barrier()

    def scale(rows_ref, ci):
        for g in range(K // L):
            wvec = w_v[ci, pl.ds(L * g, L)]
            for l in range(L):
                e = L * g + l
                wsplat = wvec.at[jnp.full((L,), l, jnp.int32)].get(
                    mode="promise_in_bounds")
                for j in range(D // L):
                    sl = pl.ds(L * j, L)
                    rows_ref[e, sl] = rows_ref[e, sl] * wsplat

    def start_gather(ci, rows_ref, gsem):
        pltpu.async_copy(support_hbm.at[src_v.at[ci]], rows_ref, gsem)

    def wait_gather(rows_ref, gsem):
        pltpu.make_async_copy(support_hbm.at[src_v.at[0]], rows_ref,
                              gsem).wait()

    def emit(rows_ref, ci):
        scale(rows_ref, ci)
        pltpu.sync_copy(rows_ref, acc_sh.at[dst_v.at[ci]], add=True)

    def sup_body(sb, scarry):
        pltpu.sync_copy(src_hbm.at[wid, sb], src_v)
        pltpu.sync_copy(dst_hbm.at[wid, sb], dst_v)
        pltpu.sync_copy(w_hbm.at[wid, sb], w_v)

        start_gather(0, rows0_v, sem0)

        def pair_body(i, carry):
            a = 2 * i
            wait_gather(rows0_v, sem0)
            start_gather(a + 1, rows1_v, sem1)
            emit(rows0_v, a)
            wait_gather(rows1_v, sem1)
            start_gather(a + 2, rows0_v, sem0)
            emit(rows1_v, a + 1)
            return carry

        lax.fori_loop(0, (SUP - 1) // 2, pair_body, 0)
        wait_gather(rows0_v, sem0)
        emit(rows0_v, SUP - 1)
        return scarry

    lax.fori_loop(0, NSUP, sup_body, 0)
    plsc.subcore_barrier()

    pltpu.sync_copy(acc_sh.at[pl.ds(s * ROWS_PER_TILE, ROWS_PER_TILE)],
                    out_hbm.at[c, pl.ds(s * ROWS_PER_TILE, ROWS_PER_TILE)])


def _sc_scatter(support, src, dst, w):
    mesh = plsc.VectorSubcoreMesh(core_axis_name="c", subcore_axis_name="s")
    fn = functools.partial(
        pl.kernel,
        mesh=mesh,
        out_type=jax.ShapeDtypeStruct((NC, NPAD, D), jnp.float32),
        scratch_types=[
            pltpu.VMEM((SUP, K), jnp.int32),
            pltpu.VMEM((SUP, K), jnp.int32),
            pltpu.VMEM((SUP, K), jnp.float32),
            pltpu.VMEM((K, D), jnp.float32),
            pltpu.VMEM((K, D), jnp.float32),
            pltpu.VMEM((ZROWS, D), jnp.float32),
            pltpu.VMEM_SHARED((NPAD, D), jnp.float32),
            pltpu.SemaphoreType.DMA,
            pltpu.SemaphoreType.DMA,
        ],
    )(_sc_body)
    return fn(support, src, dst, w)


def kernel(input, edge_index, edge_weight, weight):
    support = _matmul(input, weight)
    src = edge_index[0].reshape(NW, NSUP, SUP, K)
    dst = edge_index[1].reshape(NW, NSUP, SUP, K)
    w = edge_weight.reshape(NW, NSUP, SUP, K)
    partial = _sc_scatter(support, src, dst, w)
    return _combine(partial)
